# baseline (speedup 1.0000x reference)
import functools

import jax
import jax.numpy as jnp
from jax import lax
from jax.experimental import pallas as pl
from jax.experimental.pallas import tpu as pltpu
from jax.experimental.pallas import tpu_sc as plsc

N = 10000
E = 160000
K = 30
G = 8

L = 16
NT = 16
NP = 10240
NPT = NP // NT
ET = E // NT
ETP = 10240
CH = 2048
NCH = ETP // CH
FH = 16

f32 = jnp.float32
i32 = jnp.int32


def _rsqrt16(d):
  x = jnp.maximum(d, 1e-12)
  i = lax.bitcast_convert_type(x, i32)
  i = jnp.int32(0x5F3759DF) - lax.shift_right_logical(i, 1)
  y = lax.bitcast_convert_type(i, f32)
  for _ in range(3):
    y = y * (1.5 - 0.5 * x * y * y)
  return jnp.where(d > 0.0, y, 0.0)


def _sc_mesh(num_cores):
  return plsc.VectorSubcoreMesh(
      core_axis_name="c", subcore_axis_name="s", num_cores=num_cores
  )


CHN = 1280
NCHN = ETP // CHN


@functools.partial(
    pl.kernel,
    out_type=jax.ShapeDtypeStruct((NT, NCHN, CHN, L), f32),
    mesh=_sc_mesh(1),
    compiler_params=pltpu.CompilerParams(use_tc_tiling_on_sc=False),
    scratch_types=[
        pltpu.VMEM_SHARED((NP, L), f32),
        pltpu.VMEM((NCHN, CHN), i32),
        pltpu.VMEM((NCHN, CHN), i32),
        pltpu.VMEM((NCHN, CHN), f32),
        pltpu.VMEM((CHN, L), f32),
        pltpu.VMEM((CHN, L), f32),
        pltpu.VMEM((CHN, L), f32),
    ],
)
def _norm_kernel(row_h, col_h, w_h, out_h, deg_sh, row_t, col_t, w_t,
                 wrep, rbuf, cbuf):
  sid = lax.axis_index("s")
  nsl = pl.ds(sid * NPT, NPT)
  pltpu.sync_copy(row_h.at[sid], row_t)
  pltpu.sync_copy(col_h.at[sid], col_t)
  pltpu.sync_copy(w_h.at[sid], w_t)

  @pl.loop(0, NPT)
  def _(i):
    wrep[i, :] = jnp.zeros((L,), f32)

  pltpu.sync_copy(wrep.at[pl.ds(0, NPT)], deg_sh.at[nsl, :])
  plsc.subcore_barrier()

  @pl.loop(0, NCHN)
  def _(c):
    @pl.loop(0, CHN // L)
    def _(g):
      gb = g * L
      wv = w_t[c, pl.ds(gb, L)]
      for e in range(L):
        wrep[gb + e, :] = jnp.broadcast_to(wv[e], (L,))

    pltpu.sync_copy(wrep, deg_sh.at[col_t.at[c]], add=True)

  plsc.subcore_barrier()
  pltpu.sync_copy(deg_sh.at[nsl, :], wrep.at[pl.ds(0, NPT)])

  @pl.loop(0, NPT)
  def _(i):
    wrep[i, :] = _rsqrt16(wrep[i, :])

  pltpu.sync_copy(wrep.at[pl.ds(0, NPT)], deg_sh.at[nsl, :])
  plsc.subcore_barrier()

  @pl.loop(0, NCHN)
  def _(c):
    pltpu.sync_copy(deg_sh.at[row_t.at[c]], rbuf)
    pltpu.sync_copy(deg_sh.at[col_t.at[c]], cbuf)

    @pl.loop(0, CHN // L)
    def _(g):
      gb = g * L
      wv = w_t[c, pl.ds(gb, L)]
      for e in range(L):
        wrep[gb + e, :] = rbuf[gb + e, :] * cbuf[gb + e, :] * wv[e]

    pltpu.sync_copy(wrep, out_h.at[sid, c])


def _make_prop(ncores):
  fp = ncores * FH

  @functools.partial(
      pl.kernel,
      out_type=jax.ShapeDtypeStruct((NP, (K + 1) * fp), f32),
      mesh=_sc_mesh(ncores),
      compiler_params=pltpu.CompilerParams(use_tc_tiling_on_sc=False),
      scratch_types=[
          pltpu.VMEM_SHARED((2, NP, FH), f32),
          pltpu.VMEM((NCH, CH), i32),
          pltpu.VMEM((NCH, CH), i32),
          pltpu.VMEM((NCH, CH), f32),
          pltpu.VMEM((CH, FH), f32),
          pltpu.VMEM((CH, FH), f32),
          pltpu.VMEM((NPT, FH), f32),
          pltpu.SemaphoreType.DMA,
          pltpu.SemaphoreType.DMA,
          pltpu.SemaphoreType.DMA,
          pltpu.SemaphoreType.DMA,
      ],
  )
  def prop(x_h, row_h, col_h, norm_h, p_h, h_sh, row_t, col_t, norm_t,
           gbuf0, gbuf1, zbuf, gsem0, gsem1, ssem0, ssem1):
    cid = lax.axis_index("c")
    sid = lax.axis_index("s")
    nsl = pl.ds(sid * NPT, NPT)
    fsl = pl.ds(cid * FH, FH)
    gbufs = (gbuf0, gbuf1)
    gsems = (gsem0, gsem1)
    ssems = (ssem0, ssem1)
    pltpu.sync_copy(row_h.at[sid], row_t)
    pltpu.sync_copy(col_h.at[sid], col_t)
    pltpu.sync_copy(norm_h.at[sid], norm_t)

    @pl.loop(0, NPT)
    def _(i):
      zbuf[i, :] = jnp.zeros((L,), f32)

    pltpu.sync_copy(x_h.at[nsl, fsl], h_sh.at[0, nsl, :])
    pltpu.sync_copy(h_sh.at[0, nsl, :], p_h.at[nsl, fsl])

    def gather_desc(src, c, b):
      return pltpu.make_async_copy(
          h_sh.at[src].at[row_t.at[c]], gbufs[b], gsems[b]
      )

    def scatter_desc(dst, c, b):
      return pltpu.make_async_copy(
          gbufs[b], h_sh.at[dst].at[col_t.at[c]], ssems[b]
      )

    def scale(c, b):
      buf = gbufs[b]

      @pl.loop(0, CH // L)
      def _(g):
        gb = g * L
        nv = norm_t[c, pl.ds(gb, L)]
        for e in range(L):
          buf[gb + e, :] = buf[gb + e, :] * nv[e]

    @pl.loop(1, K + 1)
    def _(k):
      dst = lax.rem(k, 2)
      src = 1 - dst
      pltpu.sync_copy(zbuf, h_sh.at[dst, nsl, :])
      plsc.subcore_barrier()
      gather_desc(src, 0, 0).start()

      for c in range(NCH):
        b = c % 2
        if c > 0:
          scatter_desc(dst, c - 1, 1 - b).wait()
        if c + 1 < NCH:
          gather_desc(src, c + 1, 1 - b).start()
        gather_desc(src, c, b).wait()
        scale(c, b)
        scatter_desc(dst, c, b).start(add=True)
      scatter_desc(dst, NCH - 1, (NCH - 1) % 2).wait()
      plsc.subcore_barrier()
      pltpu.sync_copy(h_sh.at[dst, nsl, :], p_h.at[nsl, pl.ds(k * fp + cid * FH, FH)])

  return prop


_prop1_kernel = _make_prop(1)
_prop_kernel = _make_prop(2)

FP = 2 * FH


def _matmul_relu(p, w, b, bn=1024):
  npad, d = p.shape
  fo = w.shape[1]

  def body(p_ref, w_ref, b_ref, o_ref):
    acc = jnp.dot(p_ref[...], w_ref[...], preferred_element_type=f32)
    o_ref[...] = jnp.maximum(acc + b_ref[...], 0.0)

  return pl.pallas_call(
      body,
      grid=(npad // bn,),
      in_specs=[
          pl.BlockSpec((bn, d), lambda i: (i, 0)),
          pl.BlockSpec((d, fo), lambda i: (0, 0)),
          pl.BlockSpec((1, fo), lambda i: (0, 0)),
      ],
      out_specs=pl.BlockSpec((bn, fo), lambda i: (i, 0)),
      out_shape=jax.ShapeDtypeStruct((npad, fo), f32),
  )(p, w, b)


def _mlp_pool(x0c, x2, x3, x4, x5, oh, l1a, l1b, l1c, l1d, l1e,
              b1, l2, b2, l3, b3, l4, b4, l5, b5):
  def body(x0_r, x2_r, x3_r, x4_r, x5_r, oh_r, l1a_r, l1b_r, l1c_r, l1d_r,
           l1e_r, b1_r, l2_r, b2_r, l3_r, b3_r, l4_r, b4_r, l5_r, b5_r, o_r):
    mm = lambda a, w: jnp.dot(a, w, preferred_element_type=f32)
    h = (mm(x0_r[...], l1a_r[...]) + mm(x2_r[...], l1b_r[...])
         + mm(x3_r[...], l1c_r[...]) + mm(x4_r[...], l1d_r[...])
         + mm(x5_r[...], l1e_r[...]))
    h = jnp.maximum(h + b1_r[...], 0.0)
    h = jnp.maximum(mm(h, l2_r[...]) + b2_r[...], 0.0)
    h = jnp.maximum(mm(h, l3_r[...]) + b3_r[...], 0.0)
    h = jnp.maximum(mm(h, l4_r[...]) + b4_r[...], 0.0)
    h = jnp.maximum(mm(h, l5_r[...]) + b5_r[...], 0.0)
    t = oh_r[...] * h
    sums = jnp.sum(t, axis=0, keepdims=True)
    cnt = jnp.sum(oh_r[...], axis=0, keepdims=True)
    o_r[...] = jnp.maximum(sums / jnp.maximum(cnt, 1.0), 0.0)

  return pl.pallas_call(
      body,
      out_shape=jax.ShapeDtypeStruct((1, G), f32),
  )(x0c, x2, x3, x4, x5, oh, l1a, l1b, l1c, l1d, l1e,
    b1, l2, b2, l3, b3, l4, b4, l5, b5)


def kernel(x, edge_index, edge_weight, batch, W1, b1, W2, b2, W3, b3, W4, b4,
           L1, bl1, L2, bl2, L3, bl3, L4, bl4, L5, bl5):
  x = x.astype(f32)
  ew = edge_weight.astype(f32)
  pad_idx = (jnp.arange(ETP - ET, dtype=i32) * 53) % N

  def pad_edges(a, fill):
    a2 = a.reshape(NT, ET)
    padv = jnp.broadcast_to(fill, (NT, ETP - ET)).astype(a.dtype)
    return jnp.concatenate([a2, padv], axis=1)

  rowp = pad_edges(edge_index[0], pad_idx)
  colp = pad_edges(edge_index[1], pad_idx)
  wp = pad_edges(ew, jnp.float32(0.0))
  row3 = rowp.reshape(NT, NCH, CH)
  col3 = colp.reshape(NT, NCH, CH)

  norm4 = _norm_kernel(
      rowp.reshape(NT, NCHN, CHN),
      colp.reshape(NT, NCHN, CHN),
      wp.reshape(NT, NCHN, CHN),
  )
  norm3 = norm4[..., 0].reshape(NT, NCH, CH)

  xp1 = jnp.zeros((NP, FH), f32).at[:N, :].set(x[:, None])
  p1 = _prop1_kernel(xp1, row3, col3, norm3)
  w1b = (jnp.zeros((K + 1, FH, FP), f32)
         .at[:, 0, :20].set(W1)
         .reshape((K + 1) * FH, FP))
  b1b = jnp.zeros((1, FP), f32).at[0, :20].set(b1)
  x2 = _matmul_relu(p1, w1b, b1b)

  def wide_w(wl, bl, fo):
    wb = (jnp.zeros((K + 1, FP, FP), f32)
          .at[:, :20, :fo].set(wl.reshape(K + 1, 20, fo))
          .reshape((K + 1) * FP, FP))
    bb = jnp.zeros((1, FP), f32).at[0, :fo].set(bl)
    return wb, bb

  p2 = _prop_kernel(x2, row3, col3, norm3)
  w2b, b2b = wide_w(W2, b2, 20)
  x3 = _matmul_relu(p2, w2b, b2b)

  p3 = _prop_kernel(x3, row3, col3, norm3)
  w3b, b3b = wide_w(W3, b3, 20)
  x4 = _matmul_relu(p3, w3b, b3b)

  p4 = _prop_kernel(x4, row3, col3, norm3)
  w4b, b4b = wide_w(W4, b4, 19)
  x5 = _matmul_relu(p4, w4b, b4b)

  batch_p = jnp.concatenate([batch, jnp.full((NP - N,), G, jnp.int32)])
  oh = (batch_p[:, None] == jnp.arange(G, dtype=jnp.int32)[None, :]).astype(f32)

  def pad_rows(a, rows):
    return jnp.zeros((rows, a.shape[1]), f32).at[: a.shape[0]].set(a)

  l1a = jnp.zeros((FH, 60), f32).at[0].set(L1[0])
  out = _mlp_pool(
      xp1, x2, x3, x4, x5, oh,
      l1a,
      pad_rows(L1[1:21], FP), pad_rows(L1[21:41], FP),
      pad_rows(L1[41:61], FP), pad_rows(L1[61:80], FP),
      bl1.reshape(1, -1), L2, bl2.reshape(1, -1), L3, bl3.reshape(1, -1),
      L4, bl4.reshape(1, -1), L5, bl5.reshape(1, -1),
  )
  return out.reshape(G)


# speedup vs baseline: 1.2072x; 1.2072x over previous
import functools

import jax
import jax.numpy as jnp
from jax import lax
from jax.experimental import pallas as pl
from jax.experimental.pallas import tpu as pltpu
from jax.experimental.pallas import tpu_sc as plsc

N = 10000
E = 160000
K = 30
G = 8

L = 16
NT = 16
NP = 10240
NPT = NP // NT
ET = E // NT
ETP = 10240
CH = 2048
NCH = ETP // CH
FH = 16

f32 = jnp.float32
i32 = jnp.int32


def _rsqrt16(d):
  x = jnp.maximum(d, 1e-12)
  i = lax.bitcast_convert_type(x, i32)
  i = jnp.int32(0x5F3759DF) - lax.shift_right_logical(i, 1)
  y = lax.bitcast_convert_type(i, f32)
  for _ in range(3):
    y = y * (1.5 - 0.5 * x * y * y)
  return jnp.where(d > 0.0, y, 0.0)


def _sc_mesh(num_cores):
  return plsc.VectorSubcoreMesh(
      core_axis_name="c", subcore_axis_name="s", num_cores=num_cores
  )


CHN = 1280
NCHN = ETP // CHN


@functools.partial(
    pl.kernel,
    out_type=jax.ShapeDtypeStruct((NT, NCHN, CHN, L), f32),
    mesh=_sc_mesh(1),
    compiler_params=pltpu.CompilerParams(use_tc_tiling_on_sc=False),
    scratch_types=[
        pltpu.VMEM_SHARED((NP, L), f32),
        pltpu.VMEM((NCHN, CHN), i32),
        pltpu.VMEM((NCHN, CHN), i32),
        pltpu.VMEM((NCHN, CHN), f32),
        pltpu.VMEM((CHN, L), f32),
        pltpu.VMEM((CHN, L), f32),
        pltpu.VMEM((CHN, L), f32),
    ],
)
def _norm_kernel(row_h, col_h, w_h, out_h, deg_sh, row_t, col_t, w_t,
                 wrep, rbuf, cbuf):
  sid = lax.axis_index("s")
  nsl = pl.ds(sid * NPT, NPT)
  pltpu.sync_copy(row_h.at[sid], row_t)
  pltpu.sync_copy(col_h.at[sid], col_t)
  pltpu.sync_copy(w_h.at[sid], w_t)

  @pl.loop(0, NPT)
  def _(i):
    wrep[i, :] = jnp.zeros((L,), f32)

  pltpu.sync_copy(wrep.at[pl.ds(0, NPT)], deg_sh.at[nsl, :])
  plsc.subcore_barrier()

  @pl.loop(0, NCHN)
  def _(c):
    @pl.loop(0, CHN // L)
    def _(g):
      gb = g * L
      wv = w_t[c, pl.ds(gb, L)]
      for e in range(L):
        wrep[gb + e, :] = jnp.broadcast_to(wv[e], (L,))

    pltpu.sync_copy(wrep, deg_sh.at[col_t.at[c]], add=True)

  plsc.subcore_barrier()
  pltpu.sync_copy(deg_sh.at[nsl, :], wrep.at[pl.ds(0, NPT)])

  @pl.loop(0, NPT)
  def _(i):
    wrep[i, :] = _rsqrt16(wrep[i, :])

  pltpu.sync_copy(wrep.at[pl.ds(0, NPT)], deg_sh.at[nsl, :])
  plsc.subcore_barrier()

  @pl.loop(0, NCHN)
  def _(c):
    pltpu.sync_copy(deg_sh.at[row_t.at[c]], rbuf)
    pltpu.sync_copy(deg_sh.at[col_t.at[c]], cbuf)

    @pl.loop(0, CHN // L)
    def _(g):
      gb = g * L
      wv = w_t[c, pl.ds(gb, L)]
      for e in range(L):
        wrep[gb + e, :] = rbuf[gb + e, :] * cbuf[gb + e, :] * wv[e]

    pltpu.sync_copy(wrep, out_h.at[sid, c])


def _make_prop(ncores):
  fp = ncores * FH

  @functools.partial(
      pl.kernel,
      out_type=jax.ShapeDtypeStruct((NP, (K + 1) * fp), f32),
      mesh=_sc_mesh(ncores),
      compiler_params=pltpu.CompilerParams(use_tc_tiling_on_sc=False),
      scratch_types=[
          pltpu.VMEM_SHARED((2, NP, FH), f32),
          pltpu.VMEM((NCH, CH), i32),
          pltpu.VMEM((NCH, CH), i32),
          pltpu.VMEM((NCH, CH), f32),
          pltpu.VMEM((CH, FH), f32),
          pltpu.VMEM((CH, FH), f32),
          pltpu.VMEM((NPT, FH), f32),
          pltpu.SemaphoreType.DMA,
          pltpu.SemaphoreType.DMA,
          pltpu.SemaphoreType.DMA,
          pltpu.SemaphoreType.DMA,
          pltpu.SemaphoreType.DMA,
      ],
  )
  def prop(x_h, row_h, col_h, norm_h, p_h, h_sh, row_t, col_t, norm_t,
           gbuf0, gbuf1, zbuf, gsem0, gsem1, ssem0, ssem1, psem):
    cid = lax.axis_index("c")
    sid = lax.axis_index("s")
    nsl = pl.ds(sid * NPT, NPT)
    fsl = pl.ds(cid * FH, FH)
    gbufs = (gbuf0, gbuf1)
    gsems = (gsem0, gsem1)
    ssems = (ssem0, ssem1)
    pltpu.sync_copy(row_h.at[sid], row_t)
    pltpu.sync_copy(col_h.at[sid], col_t)
    pltpu.sync_copy(norm_h.at[sid], norm_t)

    @pl.loop(0, NPT)
    def _(i):
      zbuf[i, :] = jnp.zeros((L,), f32)

    pltpu.sync_copy(x_h.at[nsl, fsl], h_sh.at[0, nsl, :])
    pltpu.sync_copy(h_sh.at[0, nsl, :], p_h.at[nsl, fsl])

    def gather_desc(src, c, b):
      return pltpu.make_async_copy(
          h_sh.at[src].at[row_t.at[c]], gbufs[b], gsems[b]
      )

    def scatter_desc(dst, c, b):
      return pltpu.make_async_copy(
          gbufs[b], h_sh.at[dst].at[col_t.at[c]], ssems[b]
      )

    def pwrite_desc(slot, k):
      return pltpu.make_async_copy(
          h_sh.at[slot, nsl, :],
          p_h.at[nsl, pl.ds(k * fp + cid * FH, FH)],
          psem,
      )

    def scale(c, b):
      buf = gbufs[b]

      @plsc.parallel_loop(0, CH // L, unroll=2)
      def _(g):
        gb = g * L
        nv = norm_t[c, pl.ds(gb, L)]
        for e in range(L):
          buf[gb + e, :] = buf[gb + e, :] * nv[e]

    @pl.loop(1, K + 1)
    def _(k):
      dst = lax.rem(k, 2)
      src = 1 - dst
      pltpu.sync_copy(zbuf, h_sh.at[dst, nsl, :])
      plsc.subcore_barrier()
      gather_desc(src, 0, 0).start()

      for c in range(NCH):
        b = c % 2
        if c > 0:
          scatter_desc(dst, c - 1, 1 - b).wait()
        if c + 1 < NCH:
          gather_desc(src, c + 1, 1 - b).start()
        gather_desc(src, c, b).wait()
        scale(c, b)
        scatter_desc(dst, c, b).start(add=True)
      scatter_desc(dst, NCH - 1, (NCH - 1) % 2).wait()
      plsc.subcore_barrier()
      @pl.when(k > 1)
      def _():
        pwrite_desc(1 - dst, k - 1).wait()

      pwrite_desc(dst, k).start()

    pwrite_desc(lax.rem(jnp.int32(K), 2), jnp.int32(K)).wait()

  return prop


_prop1_kernel = _make_prop(1)
_prop_kernel = _make_prop(2)

FP = 2 * FH


def _matmul_relu(p, w, b, bn=1024):
  npad, d = p.shape
  fo = w.shape[1]

  def body(p_ref, w_ref, b_ref, o_ref):
    acc = jnp.dot(p_ref[...], w_ref[...], preferred_element_type=f32)
    o_ref[...] = jnp.maximum(acc + b_ref[...], 0.0)

  return pl.pallas_call(
      body,
      grid=(npad // bn,),
      in_specs=[
          pl.BlockSpec((bn, d), lambda i: (i, 0)),
          pl.BlockSpec((d, fo), lambda i: (0, 0)),
          pl.BlockSpec((1, fo), lambda i: (0, 0)),
      ],
      out_specs=pl.BlockSpec((bn, fo), lambda i: (i, 0)),
      out_shape=jax.ShapeDtypeStruct((npad, fo), f32),
  )(p, w, b)


def _mlp_pool(x0c, x2, x3, x4, x5, oh, l1a, l1b, l1c, l1d, l1e,
              b1, l2, b2, l3, b3, l4, b4, l5, b5):
  def body(x0_r, x2_r, x3_r, x4_r, x5_r, oh_r, l1a_r, l1b_r, l1c_r, l1d_r,
           l1e_r, b1_r, l2_r, b2_r, l3_r, b3_r, l4_r, b4_r, l5_r, b5_r, o_r):
    mm = lambda a, w: jnp.dot(a, w, preferred_element_type=f32)
    h = (mm(x0_r[...], l1a_r[...]) + mm(x2_r[...], l1b_r[...])
         + mm(x3_r[...], l1c_r[...]) + mm(x4_r[...], l1d_r[...])
         + mm(x5_r[...], l1e_r[...]))
    h = jnp.maximum(h + b1_r[...], 0.0)
    h = jnp.maximum(mm(h, l2_r[...]) + b2_r[...], 0.0)
    h = jnp.maximum(mm(h, l3_r[...]) + b3_r[...], 0.0)
    h = jnp.maximum(mm(h, l4_r[...]) + b4_r[...], 0.0)
    h = jnp.maximum(mm(h, l5_r[...]) + b5_r[...], 0.0)
    t = oh_r[...] * h
    sums = jnp.sum(t, axis=0, keepdims=True)
    cnt = jnp.sum(oh_r[...], axis=0, keepdims=True)
    o_r[...] = jnp.maximum(sums / jnp.maximum(cnt, 1.0), 0.0)

  return pl.pallas_call(
      body,
      out_shape=jax.ShapeDtypeStruct((1, G), f32),
  )(x0c, x2, x3, x4, x5, oh, l1a, l1b, l1c, l1d, l1e,
    b1, l2, b2, l3, b3, l4, b4, l5, b5)


def kernel(x, edge_index, edge_weight, batch, W1, b1, W2, b2, W3, b3, W4, b4,
           L1, bl1, L2, bl2, L3, bl3, L4, bl4, L5, bl5):
  x = x.astype(f32)
  ew = edge_weight.astype(f32)
  pad_idx = (jnp.arange(ETP - ET, dtype=i32) * 53) % N

  def pad_edges(a, fill):
    a2 = a.reshape(NT, ET)
    padv = jnp.broadcast_to(fill, (NT, ETP - ET)).astype(a.dtype)
    return jnp.concatenate([a2, padv], axis=1)

  rowp = pad_edges(edge_index[0], pad_idx)
  colp = pad_edges(edge_index[1], pad_idx)
  wp = pad_edges(ew, jnp.float32(0.0))
  row3 = rowp.reshape(NT, NCH, CH)
  col3 = colp.reshape(NT, NCH, CH)

  norm4 = _norm_kernel(
      rowp.reshape(NT, NCHN, CHN),
      colp.reshape(NT, NCHN, CHN),
      wp.reshape(NT, NCHN, CHN),
  )
  norm3 = norm4[..., 0].reshape(NT, NCH, CH)

  xp1 = jnp.zeros((NP, FH), f32).at[:N, :].set(x[:, None])
  p1 = _prop1_kernel(xp1, row3, col3, norm3)
  w1b = (jnp.zeros((K + 1, FH, FP), f32)
         .at[:, 0, :20].set(W1)
         .reshape((K + 1) * FH, FP))
  b1b = jnp.zeros((1, FP), f32).at[0, :20].set(b1)
  x2 = _matmul_relu(p1, w1b, b1b)

  def wide_w(wl, bl, fo):
    wb = (jnp.zeros((K + 1, FP, FP), f32)
          .at[:, :20, :fo].set(wl.reshape(K + 1, 20, fo))
          .reshape((K + 1) * FP, FP))
    bb = jnp.zeros((1, FP), f32).at[0, :fo].set(bl)
    return wb, bb

  p2 = _prop_kernel(x2, row3, col3, norm3)
  w2b, b2b = wide_w(W2, b2, 20)
  x3 = _matmul_relu(p2, w2b, b2b)

  p3 = _prop_kernel(x3, row3, col3, norm3)
  w3b, b3b = wide_w(W3, b3, 20)
  x4 = _matmul_relu(p3, w3b, b3b)

  p4 = _prop_kernel(x4, row3, col3, norm3)
  w4b, b4b = wide_w(W4, b4, 19)
  x5 = _matmul_relu(p4, w4b, b4b)

  batch_p = jnp.concatenate([batch, jnp.full((NP - N,), G, jnp.int32)])
  oh = (batch_p[:, None] == jnp.arange(G, dtype=jnp.int32)[None, :]).astype(f32)

  def pad_rows(a, rows):
    return jnp.zeros((rows, a.shape[1]), f32).at[: a.shape[0]].set(a)

  l1a = jnp.zeros((FH, 60), f32).at[0].set(L1[0])
  out = _mlp_pool(
      xp1, x2, x3, x4, x5, oh,
      l1a,
      pad_rows(L1[1:21], FP), pad_rows(L1[21:41], FP),
      pad_rows(L1[41:61], FP), pad_rows(L1[61:80], FP),
      bl1.reshape(1, -1), L2, bl2.reshape(1, -1), L3, bl3.reshape(1, -1),
      L4, bl4.reshape(1, -1), L5, bl5.reshape(1, -1),
  )
  return out.reshape(G)
